# baseline (speedup 1.0000x reference)
import jax
import jax.numpy as jnp
from jax import lax
from jax.experimental import pallas as pl
from jax.experimental.pallas import tpu as pltpu
from jax.experimental.pallas import tpu_sc as plsc

B = 1024
HIST = 200
D = 32
NSAMP = 64
NC, NS = 2, 16
NW = NC * NS
BW = B // NW
LP = 208
NBLK = LP // 16
S1 = 104
S2 = HIST - S1
VW = 128


def _sc_body(dtq_hbm, dtc_hbm, tlq_hbm, tlc_hbm, llq_hbm, llc_hbm,
             wordv_hbm, docv_hbm, labv_hbm,
             outdoc_hbm, labout_hbm, nzout_hbm,
             dtq_v, dtc_v, tlq_v, tlc_v, llq_v, llc_v,
             w_v, d_v, lab_v, labpack_v, nzpack_v, out_v, wts_v,
             sem1, sem2, semd):
    wid = lax.axis_index("c") * NS + lax.axis_index("s")
    base = wid * BW
    dbase = wid * (BW + NSAMP)
    lane = lax.broadcasted_iota(jnp.int32, (16,), 0)
    lane16 = lane + 16

    pltpu.sync_copy(dtq_hbm.at[pl.ds(pl.multiple_of(base * HIST, 8), BW * HIST)],
                    dtq_v)
    pltpu.sync_copy(dtc_hbm.at[pl.ds(pl.multiple_of(base * HIST, 8), BW * HIST)],
                    dtc_v)
    pltpu.sync_copy(tlq_hbm, tlq_v)
    pltpu.sync_copy(tlc_hbm, tlc_v)
    pltpu.sync_copy(llq_hbm, llq_v)
    pltpu.sync_copy(llc_hbm, llc_v)

    pltpu.async_copy(
        docv_hbm.at[tlq_v.at[pl.ds(pl.multiple_of(dbase, 8), BW + NSAMP)]],
        d_v, semd).wait()
    pltpu.async_copy(labv_hbm.at[llq_v.at[pl.ds(pl.multiple_of(base, 8), BW)]],
                     lab_v, semd).wait()

    for r in range(BW):
        rfull = jnp.full((16,), r, jnp.int32)
        cb = plsc.load_gather(llc_v, [jnp.full((16,), base + r, jnp.int32)])
        v0 = plsc.load_gather(lab_v, [rfull, cb + lane])
        v1 = plsc.load_gather(lab_v, [rfull, cb + lane16])
        plsc.store_scatter(labpack_v, [(r * D) + lane], v0)
        plsc.store_scatter(labpack_v, [(r * D) + lane16], v1)
    pltpu.sync_copy(labpack_v,
                    labout_hbm.at[pl.ds(pl.multiple_of(base * D, 8), BW * D)])

    for r in range(NSAMP):
        rfull = jnp.full((16,), BW + r, jnp.int32)
        cb = plsc.load_gather(tlc_v, [jnp.full((16,), dbase + BW + r, jnp.int32)])
        v0 = plsc.load_gather(d_v, [rfull, cb + lane])
        v1 = plsc.load_gather(d_v, [rfull, cb + lane16])
        plsc.store_scatter(nzpack_v, [(r * D) + lane], v0)
        plsc.store_scatter(nzpack_v, [(r * D) + lane16], v1)

    @pl.when(lax.axis_index("s") == 0)
    def _():
        half = NSAMP * D // 2
        hoff = pl.multiple_of(lax.axis_index("c") * half, 8)
        pltpu.sync_copy(nzpack_v.at[pl.ds(hoff, half)],
                        nzout_hbm.at[pl.ds(hoff, half)])

    rows_c = [blk * 16 + lane for blk in range(NBLK)]
    tail_mask = lane < (HIST - 16 * (NBLK - 1))

    def b_body(b, carry):
        off = pl.multiple_of(b * HIST, 8)
        cp1 = pltpu.async_copy(wordv_hbm.at[dtq_v.at[pl.ds(off, S1)]],
                               w_v.at[pl.ds(0, S1)], sem1)
        cp2 = pltpu.async_copy(wordv_hbm.at[dtq_v.at[pl.ds(off + S1, S2)]],
                               w_v.at[pl.ds(S1, S2)], sem2)
        cp1.wait()
        cp2.wait()

        bfull = jnp.full((16,), b, jnp.int32)
        offv = jnp.full((16,), off, jnp.int32)
        cbs = [plsc.load_gather(dtc_v, [offv + (blk * 16) + lane])
               for blk in range(NBLK)]
        tlcb = plsc.load_gather(tlc_v, [jnp.full((16,), dbase, jnp.int32) + bfull])

        def j_body(j, sc):
            dj = plsc.load_gather(d_v, [bfull, tlcb + j])
            return tuple(sc[k] + plsc.load_gather(w_v, [rows_c[k], cbs[k] + j]) * dj
                         for k in range(NBLK))

        sc0 = tuple(jnp.zeros((16,), jnp.float32) for _ in range(NBLK))
        sc = list(lax.fori_loop(0, D, j_body, sc0))
        sc[NBLK - 1] = jnp.where(tail_mask, sc[NBLK - 1], -1e30)

        m = sc[0]
        for k in range(1, NBLK):
            m = jnp.maximum(m, sc[k])
        mm = jnp.max(m)
        es = [jnp.exp(s - mm) for s in sc]
        tot = es[0]
        for k in range(1, NBLK):
            tot = tot + es[k]
        inv = 1.0 / jnp.full((16,), jnp.sum(tot), jnp.float32)
        for k in range(NBLK):
            plsc.store_scatter(wts_v, [(k * 16) + lane], es[k] * inv)

        def l_body(i, acc):
            a0, a1 = acc
            for u in range(4):
                l = i * 4 + u
                lfull = jnp.full((16,), l, jnp.int32)
                wt = plsc.load_gather(wts_v, [lfull])
                cb = plsc.load_gather(dtc_v, [offv + lfull])
                a0 = a0 + wt * plsc.load_gather(w_v, [lfull, cb + lane])
                a1 = a1 + wt * plsc.load_gather(w_v, [lfull, cb + lane16])
            return (a0, a1)

        a0, a1 = lax.fori_loop(0, HIST // 4, l_body,
                               (jnp.zeros((16,), jnp.float32),
                                jnp.zeros((16,), jnp.float32)))
        bd = b * D
        plsc.store_scatter(out_v, [bd + lane], a0)
        plsc.store_scatter(out_v, [bd + lane16], a1)
        return carry

    lax.fori_loop(0, BW, b_body, 0)
    pltpu.sync_copy(out_v,
                    outdoc_hbm.at[pl.ds(pl.multiple_of(base * D, 8), BW * D)])


def _sc_call(dtq, dtc, tlq, tlc, llq, llc, wordv, docv, labv):
    mesh = plsc.VectorSubcoreMesh(core_axis_name="c", subcore_axis_name="s",
                                  num_cores=NC, num_subcores=NS)
    out_types = (jax.ShapeDtypeStruct((B * D,), jnp.float32),
                 jax.ShapeDtypeStruct((B * D,), jnp.float32),
                 jax.ShapeDtypeStruct((NSAMP * D,), jnp.float32))
    scratch = [
        pltpu.VMEM((BW * HIST,), jnp.int32),
        pltpu.VMEM((BW * HIST,), jnp.int32),
        pltpu.VMEM((NW * (BW + NSAMP),), jnp.int32),
        pltpu.VMEM((NW * (BW + NSAMP),), jnp.int32),
        pltpu.VMEM((B,), jnp.int32),
        pltpu.VMEM((B,), jnp.int32),
        pltpu.VMEM((LP, VW), jnp.float32),
        pltpu.VMEM((BW + NSAMP, VW), jnp.float32),
        pltpu.VMEM((BW, VW), jnp.float32),
        pltpu.VMEM((BW * D,), jnp.float32),
        pltpu.VMEM((NSAMP * D,), jnp.float32),
        pltpu.VMEM((BW * D,), jnp.float32),
        pltpu.VMEM((LP,), jnp.float32),
        pltpu.SemaphoreType.DMA,
        pltpu.SemaphoreType.DMA,
        pltpu.SemaphoreType.DMA,
    ]
    k = pl.kernel(_sc_body, out_type=out_types, mesh=mesh,
                  scratch_types=scratch,
                  compiler_params=pltpu.CompilerParams(
                      needs_layout_passes=False))
    return k(dtq, dtc, tlq, tlc, llq, llc, wordv, docv, labv)


def _score_body(y_ref, x0_ref, lab_ref, out_ref):
    y = y_ref[...]
    lab = lab_ref[...]
    x0 = x0_ref[...]
    dn = (((1,), (1,)), ((), ()))
    s = lax.dot_general(y, lab, dn, preferred_element_type=jnp.float32)
    s0 = lax.dot_general(x0, lab, dn, preferred_element_type=jnp.float32)
    out_ref[...] = jnp.maximum(s - s0 + 1.0, 0.0)


def kernel(dt, tl, ll, num_sampled, opt, noise_ids, word_table, doc_table,
           label_table):
    del num_sampled, opt
    wordv = word_table.reshape(-1, VW)
    docv = doc_table.reshape(-1, VW)
    labv = label_table.reshape(-1, VW)
    dtf = dt.reshape(-1)
    tlq2 = jnp.concatenate(
        [(tl >> 2).reshape(NW, BW),
         jnp.broadcast_to(noise_ids >> 2, (NW, NSAMP))], axis=1).reshape(-1)
    tlc2 = jnp.concatenate(
        [((tl & 3) << 5).reshape(NW, BW),
         jnp.broadcast_to((noise_ids & 3) << 5, (NW, NSAMP))], axis=1).reshape(-1)
    outdoc, labr, nzrows = _sc_call(
        dtf >> 2, (dtf & 3) << 5,
        tlq2, tlc2,
        ll >> 2, (ll & 3) << 5,
        wordv, docv, labv)
    outdoc = outdoc.reshape(B, D)
    labr = labr.reshape(B, D)
    nzrows = nzrows.reshape(NSAMP, D)
    y = jnp.concatenate([outdoc[1:], nzrows], axis=0)
    x0 = outdoc[0:1]
    return pl.pallas_call(
        _score_body,
        out_shape=jax.ShapeDtypeStruct((B + NSAMP - 1, B), jnp.float32),
    )(y, x0, labr)


# speedup vs baseline: 1.4108x; 1.4108x over previous
import jax
import jax.numpy as jnp
from jax import lax
from jax.experimental import pallas as pl
from jax.experimental.pallas import tpu as pltpu
from jax.experimental.pallas import tpu_sc as plsc

B = 1024
HIST = 200
D = 32
NSAMP = 64
NC, NS = 2, 16
NW = NC * NS
BW = B // NW
LP = 208
NBLK = LP // 16
S1 = 104
S2 = HIST - S1
VW = 128


def _sc_body(dtq_hbm, dtc_hbm, tlq_hbm, tlc_hbm, llq_hbm, llc_hbm,
             wordv_hbm, docv_hbm, labv_hbm,
             outdoc_hbm, labout_hbm, nzout_hbm,
             dtq_v, dtc_v, tlq_v, tlc_v, llq_v, llc_v,
             w_v, d_v, lab_v, labpack_v, nzpack_v, out_v, wts_v,
             sem1, sem2, semd):
    wid = lax.axis_index("c") * NS + lax.axis_index("s")
    base = wid * BW
    dbase = wid * (BW + NSAMP)
    lane = lax.broadcasted_iota(jnp.int32, (16,), 0)
    lane16 = lane + 16

    pltpu.sync_copy(dtq_hbm.at[pl.ds(pl.multiple_of(base * HIST, 8), BW * HIST)],
                    dtq_v)
    pltpu.sync_copy(dtc_hbm.at[pl.ds(pl.multiple_of(base * HIST, 8), BW * HIST)],
                    dtc_v)
    pltpu.sync_copy(tlq_hbm, tlq_v)
    pltpu.sync_copy(tlc_hbm, tlc_v)
    pltpu.sync_copy(llq_hbm, llq_v)
    pltpu.sync_copy(llc_hbm, llc_v)

    pltpu.async_copy(
        docv_hbm.at[tlq_v.at[pl.ds(pl.multiple_of(dbase, 8), BW + NSAMP)]],
        d_v, semd).wait()
    pltpu.async_copy(labv_hbm.at[llq_v.at[pl.ds(pl.multiple_of(base, 8), BW)]],
                     lab_v, semd).wait()

    for r in range(BW):
        rfull = jnp.full((16,), r, jnp.int32)
        cb = plsc.load_gather(llc_v, [jnp.full((16,), base + r, jnp.int32)])
        v0 = plsc.load_gather(lab_v, [rfull, cb + lane])
        v1 = plsc.load_gather(lab_v, [rfull, cb + lane16])
        plsc.store_scatter(labpack_v, [(r * D) + lane], v0)
        plsc.store_scatter(labpack_v, [(r * D) + lane16], v1)
    pltpu.sync_copy(labpack_v,
                    labout_hbm.at[pl.ds(pl.multiple_of(base * D, 8), BW * D)])

    for r in range(NSAMP):
        rfull = jnp.full((16,), BW + r, jnp.int32)
        cb = plsc.load_gather(tlc_v, [jnp.full((16,), dbase + BW + r, jnp.int32)])
        v0 = plsc.load_gather(d_v, [rfull, cb + lane])
        v1 = plsc.load_gather(d_v, [rfull, cb + lane16])
        plsc.store_scatter(nzpack_v, [(r * D) + lane], v0)
        plsc.store_scatter(nzpack_v, [(r * D) + lane16], v1)

    @pl.when(lax.axis_index("s") == 0)
    def _():
        half = NSAMP * D // 2
        hoff = pl.multiple_of(lax.axis_index("c") * half, 8)
        pltpu.sync_copy(nzpack_v.at[pl.ds(hoff, half)],
                        nzout_hbm.at[pl.ds(hoff, half)])

    rows_c = [blk * 16 + lane for blk in range(NBLK)]
    tail_mask = lane < (HIST - 16 * (NBLK - 1))

    def b_body(b, carry):
        off = pl.multiple_of(b * HIST, 8)
        cp1 = pltpu.async_copy(wordv_hbm.at[dtq_v.at[pl.ds(off, S1)]],
                               w_v.at[pl.ds(0, S1)], sem1)
        cp2 = pltpu.async_copy(wordv_hbm.at[dtq_v.at[pl.ds(off + S1, S2)]],
                               w_v.at[pl.ds(S1, S2)], sem2)
        cp1.wait()
        cp2.wait()

        bfull = jnp.full((16,), b, jnp.int32)
        offv = jnp.full((16,), off, jnp.int32)
        cbs = [plsc.load_gather(dtc_v, [offv + (blk * 16) + lane])
               for blk in range(NBLK)]
        tlcb = plsc.load_gather(tlc_v, [jnp.full((16,), dbase, jnp.int32) + bfull])

        def j_body(j, sc):
            dj = plsc.load_gather(d_v, [bfull, tlcb + j])
            return tuple(sc[k] + plsc.load_gather(w_v, [rows_c[k], cbs[k] + j]) * dj
                         for k in range(NBLK))

        sc0 = tuple(jnp.zeros((16,), jnp.float32) for _ in range(NBLK))
        sc = list(lax.fori_loop(0, D, j_body, sc0))
        sc[NBLK - 1] = jnp.where(tail_mask, sc[NBLK - 1], -1e30)

        m = sc[0]
        for k in range(1, NBLK):
            m = jnp.maximum(m, sc[k])
        mm = jnp.max(m)
        es = [jnp.exp(s - mm) for s in sc]
        tot = es[0]
        for k in range(1, NBLK):
            tot = tot + es[k]
        inv = 1.0 / jnp.full((16,), jnp.sum(tot), jnp.float32)
        for k in range(NBLK):
            plsc.store_scatter(wts_v, [(k * 16) + lane], es[k] * inv)

        def l_body(i, acc):
            a0, a1 = acc
            for u in range(4):
                l = i * 4 + u
                lfull = jnp.full((16,), l, jnp.int32)
                wt = plsc.load_gather(wts_v, [lfull])
                cb = plsc.load_gather(dtc_v, [offv + lfull])
                a0 = a0 + wt * plsc.load_gather(w_v, [lfull, cb + lane])
                a1 = a1 + wt * plsc.load_gather(w_v, [lfull, cb + lane16])
            return (a0, a1)

        a0, a1 = lax.fori_loop(0, HIST // 4, l_body,
                               (jnp.zeros((16,), jnp.float32),
                                jnp.zeros((16,), jnp.float32)))
        bd = b * D
        plsc.store_scatter(out_v, [bd + lane], a0)
        plsc.store_scatter(out_v, [bd + lane16], a1)
        return carry

    lax.fori_loop(0, BW, b_body, 0)
    pltpu.sync_copy(out_v,
                    outdoc_hbm.at[pl.ds(pl.multiple_of(base * D, 8), BW * D)])


def _sc_call(dtq, dtc, tlq, tlc, llq, llc, wordv, docv, labv):
    mesh = plsc.VectorSubcoreMesh(core_axis_name="c", subcore_axis_name="s",
                                  num_cores=NC, num_subcores=NS)
    out_types = (jax.ShapeDtypeStruct((B * D,), jnp.float32),
                 jax.ShapeDtypeStruct((B * D,), jnp.float32),
                 jax.ShapeDtypeStruct((NSAMP * D,), jnp.float32))
    scratch = [
        pltpu.VMEM((BW * HIST,), jnp.int32),
        pltpu.VMEM((BW * HIST,), jnp.int32),
        pltpu.VMEM((NW * (BW + NSAMP),), jnp.int32),
        pltpu.VMEM((NW * (BW + NSAMP),), jnp.int32),
        pltpu.VMEM((B,), jnp.int32),
        pltpu.VMEM((B,), jnp.int32),
        pltpu.VMEM((LP, VW), jnp.float32),
        pltpu.VMEM((BW + NSAMP, VW), jnp.float32),
        pltpu.VMEM((BW, VW), jnp.float32),
        pltpu.VMEM((BW * D,), jnp.float32),
        pltpu.VMEM((NSAMP * D,), jnp.float32),
        pltpu.VMEM((BW * D,), jnp.float32),
        pltpu.VMEM((LP,), jnp.float32),
        pltpu.SemaphoreType.DMA,
        pltpu.SemaphoreType.DMA,
        pltpu.SemaphoreType.DMA,
    ]
    k = pl.kernel(_sc_body, out_type=out_types, mesh=mesh,
                  scratch_types=scratch,
                  compiler_params=pltpu.CompilerParams(
                      needs_layout_passes=False))
    return k(dtq, dtc, tlq, tlc, llq, llc, wordv, docv, labv)


XB = 8192
XR = XB // 4


def _xpose_body(in_ref, out_ref):
    x = in_ref[...]
    out_ref[...] = jnp.concatenate(
        [jnp.swapaxes(x[:, p * XR:(p + 1) * XR], 0, 1) for p in range(4)],
        axis=1)


def _to_rowmajor(table):
    t_t = table.T
    v = t_t.shape[1]
    steps = -(-v // XB)
    return pl.pallas_call(
        _xpose_body,
        grid=(steps,),
        in_specs=[pl.BlockSpec((D, XB), lambda t: (0, t))],
        out_specs=pl.BlockSpec((XR, VW), lambda t: (t, 0)),
        out_shape=jax.ShapeDtypeStruct((steps * XR, VW), jnp.float32),
    )(t_t)


def _vq(i):
    return ((i >> 13) << 11) + (i & (XR - 1))


def _vc(i):
    return ((i >> 11) & 3) << 5


def _score_body(y_ref, x0_ref, lab_ref, out_ref):
    y = y_ref[...]
    lab = lab_ref[...]
    x0 = x0_ref[...]
    dn = (((1,), (1,)), ((), ()))
    s = lax.dot_general(y, lab, dn, preferred_element_type=jnp.float32)
    s0 = lax.dot_general(x0, lab, dn, preferred_element_type=jnp.float32)
    out_ref[...] = jnp.maximum(s - s0 + 1.0, 0.0)


def kernel(dt, tl, ll, num_sampled, opt, noise_ids, word_table, doc_table,
           label_table):
    del num_sampled, opt
    wordv = _to_rowmajor(word_table)
    docv = _to_rowmajor(doc_table)
    labv = _to_rowmajor(label_table)
    dtf = dt.reshape(-1)
    tlq2 = jnp.concatenate(
        [_vq(tl).reshape(NW, BW),
         jnp.broadcast_to(_vq(noise_ids), (NW, NSAMP))], axis=1).reshape(-1)
    tlc2 = jnp.concatenate(
        [_vc(tl).reshape(NW, BW),
         jnp.broadcast_to(_vc(noise_ids), (NW, NSAMP))], axis=1).reshape(-1)
    outdoc, labr, nzrows = _sc_call(
        _vq(dtf), _vc(dtf),
        tlq2, tlc2,
        _vq(ll), _vc(ll),
        wordv, docv, labv)
    outdoc = outdoc.reshape(B, D)
    labr = labr.reshape(B, D)
    nzrows = nzrows.reshape(NSAMP, D)
    y = jnp.concatenate([outdoc[1:], nzrows], axis=0)
    x0 = outdoc[0:1]
    return pl.pallas_call(
        _score_body,
        out_shape=jax.ShapeDtypeStruct((B + NSAMP - 1, B), jnp.float32),
    )(y, x0, labr)


# speedup vs baseline: 2.1145x; 1.4988x over previous
import jax
import jax.numpy as jnp
from jax import lax
from jax.experimental import pallas as pl
from jax.experimental.pallas import tpu as pltpu
from jax.experimental.pallas import tpu_sc as plsc

B = 1024
HIST = 200
D = 32
NSAMP = 64
NC, NS = 2, 16
NW = NC * NS
BW = B // NW
LP = 208
NBLK = LP // 16
S1 = 104
S2 = HIST - S1
VW = 128


def _sc_body(dtq_hbm, dtc_hbm, tlq_hbm, tlc_hbm, llq_hbm, llc_hbm,
             wordv_hbm, docv_hbm, labv_hbm,
             outdoc_hbm, labout_hbm, nzout_hbm,
             dtq_v, dtc_v, tlq_v, tlc_v, llq_v, llc_v,
             w_a, w_b, d_v, lab_v, labpack_v, nzpack_v, out_v, wts_v,
             sem1, sem2, sem3, sem4, semd):
    wid = lax.axis_index("c") * NS + lax.axis_index("s")
    base = wid * BW
    dbase = wid * (BW + NSAMP)
    lane = lax.broadcasted_iota(jnp.int32, (16,), 0)
    lane16 = lane + 16

    pltpu.sync_copy(dtq_hbm.at[pl.ds(pl.multiple_of(base * HIST, 8), BW * HIST)],
                    dtq_v)
    pltpu.sync_copy(dtc_hbm.at[pl.ds(pl.multiple_of(base * HIST, 8), BW * HIST)],
                    dtc_v)
    pltpu.sync_copy(tlq_hbm, tlq_v)
    pltpu.sync_copy(tlc_hbm, tlc_v)
    pltpu.sync_copy(llq_hbm, llq_v)
    pltpu.sync_copy(llc_hbm, llc_v)

    pltpu.async_copy(
        docv_hbm.at[tlq_v.at[pl.ds(pl.multiple_of(dbase, 8), BW + NSAMP)]],
        d_v, semd).wait()
    pltpu.async_copy(labv_hbm.at[llq_v.at[pl.ds(pl.multiple_of(base, 8), BW)]],
                     lab_v, semd).wait()

    for r in range(BW):
        rfull = jnp.full((16,), r, jnp.int32)
        cb = plsc.load_gather(llc_v, [jnp.full((16,), base + r, jnp.int32)])
        v0 = plsc.load_gather(lab_v, [rfull, cb + lane])
        v1 = plsc.load_gather(lab_v, [rfull, cb + lane16])
        plsc.store_scatter(labpack_v, [(r * D) + lane], v0)
        plsc.store_scatter(labpack_v, [(r * D) + lane16], v1)
    pltpu.sync_copy(labpack_v,
                    labout_hbm.at[pl.ds(pl.multiple_of(base * D, 8), BW * D)])

    for r in range(NSAMP):
        rfull = jnp.full((16,), BW + r, jnp.int32)
        cb = plsc.load_gather(tlc_v, [jnp.full((16,), dbase + BW + r, jnp.int32)])
        v0 = plsc.load_gather(d_v, [rfull, cb + lane])
        v1 = plsc.load_gather(d_v, [rfull, cb + lane16])
        plsc.store_scatter(nzpack_v, [(r * D) + lane], v0)
        plsc.store_scatter(nzpack_v, [(r * D) + lane16], v1)

    @pl.when(lax.axis_index("s") == 0)
    def _():
        half = NSAMP * D // 2
        hoff = pl.multiple_of(lax.axis_index("c") * half, 8)
        pltpu.sync_copy(nzpack_v.at[pl.ds(hoff, half)],
                        nzout_hbm.at[pl.ds(hoff, half)])

    rows_c = [blk * 16 + lane for blk in range(NBLK)]
    tail_mask = lane < (HIST - 16 * (NBLK - 1))

    def start_gather(b, w_v, sa, sb):
        off = pl.multiple_of(b * HIST, 8)
        pltpu.async_copy(wordv_hbm.at[dtq_v.at[pl.ds(off, S1)]],
                         w_v.at[pl.ds(0, S1)], sa)
        pltpu.async_copy(wordv_hbm.at[dtq_v.at[pl.ds(off + S1, S2)]],
                         w_v.at[pl.ds(S1, S2)], sb)

    def wait_gather(w_v, sa, sb):
        pltpu.make_async_copy(wordv_hbm.at[dtq_v.at[pl.ds(0, S1)]],
                              w_v.at[pl.ds(0, S1)], sa).wait()
        pltpu.make_async_copy(wordv_hbm.at[dtq_v.at[pl.ds(S1, S2)]],
                              w_v.at[pl.ds(S1, S2)], sb).wait()

    def attend(b, w_v):
        bfull = jnp.full((16,), b, jnp.int32)
        offv = jnp.full((16,), b * HIST, jnp.int32)
        cbs = [plsc.load_gather(dtc_v, [offv + (blk * 16) + lane])
               for blk in range(NBLK)]
        tlcb = plsc.load_gather(tlc_v, [jnp.full((16,), dbase, jnp.int32) + bfull])

        def j_body(j, sc):
            dj = plsc.load_gather(d_v, [bfull, tlcb + j])
            return tuple(sc[k] + plsc.load_gather(w_v, [rows_c[k], cbs[k] + j]) * dj
                         for k in range(NBLK))

        sc0 = tuple(jnp.zeros((16,), jnp.float32) for _ in range(NBLK))
        sc = list(lax.fori_loop(0, D, j_body, sc0))
        sc[NBLK - 1] = jnp.where(tail_mask, sc[NBLK - 1], -1e30)

        m = sc[0]
        for k in range(1, NBLK):
            m = jnp.maximum(m, sc[k])
        mm = jnp.max(m)
        es = [jnp.exp(s - mm) for s in sc]
        tot = es[0]
        for k in range(1, NBLK):
            tot = tot + es[k]
        inv = 1.0 / jnp.full((16,), jnp.sum(tot), jnp.float32)
        for k in range(NBLK):
            plsc.store_scatter(wts_v, [(k * 16) + lane], es[k] * inv)

        def l_body(i, acc):
            a0, a1 = acc
            for u in range(4):
                l = i * 4 + u
                lfull = jnp.full((16,), l, jnp.int32)
                wt = plsc.load_gather(wts_v, [lfull])
                cb = plsc.load_gather(dtc_v, [offv + lfull])
                a0 = a0 + wt * plsc.load_gather(w_v, [lfull, cb + lane])
                a1 = a1 + wt * plsc.load_gather(w_v, [lfull, cb + lane16])
            return (a0, a1)

        a0, a1 = lax.fori_loop(0, HIST // 4, l_body,
                               (jnp.zeros((16,), jnp.float32),
                                jnp.zeros((16,), jnp.float32)))
        bd = b * D
        plsc.store_scatter(out_v, [bd + lane], a0)
        plsc.store_scatter(out_v, [bd + lane16], a1)

    start_gather(0, w_a, sem1, sem2)

    def b_body(i, carry):
        b0 = i * 2
        start_gather(b0 + 1, w_b, sem3, sem4)
        wait_gather(w_a, sem1, sem2)
        attend(b0, w_a)
        start_gather(jnp.minimum(b0 + 2, BW - 1), w_a, sem1, sem2)
        wait_gather(w_b, sem3, sem4)
        attend(b0 + 1, w_b)
        return carry

    lax.fori_loop(0, BW // 2, b_body, 0)
    wait_gather(w_a, sem1, sem2)
    pltpu.sync_copy(out_v,
                    outdoc_hbm.at[pl.ds(pl.multiple_of(base * D, 8), BW * D)])


def _sc_call(dtq, dtc, tlq, tlc, llq, llc, wordv, docv, labv):
    mesh = plsc.VectorSubcoreMesh(core_axis_name="c", subcore_axis_name="s",
                                  num_cores=NC, num_subcores=NS)
    out_types = (jax.ShapeDtypeStruct((B * D,), jnp.float32),
                 jax.ShapeDtypeStruct((B * D,), jnp.float32),
                 jax.ShapeDtypeStruct((NSAMP * D,), jnp.float32))
    scratch = [
        pltpu.VMEM((BW * HIST,), jnp.int32),
        pltpu.VMEM((BW * HIST,), jnp.int32),
        pltpu.VMEM((NW * (BW + NSAMP),), jnp.int32),
        pltpu.VMEM((NW * (BW + NSAMP),), jnp.int32),
        pltpu.VMEM((B,), jnp.int32),
        pltpu.VMEM((B,), jnp.int32),
        pltpu.VMEM((LP, VW), jnp.float32),
        pltpu.VMEM((LP, VW), jnp.float32),
        pltpu.VMEM((BW + NSAMP, VW), jnp.float32),
        pltpu.VMEM((BW, VW), jnp.float32),
        pltpu.VMEM((BW * D,), jnp.float32),
        pltpu.VMEM((NSAMP * D,), jnp.float32),
        pltpu.VMEM((BW * D,), jnp.float32),
        pltpu.VMEM((LP,), jnp.float32),
        pltpu.SemaphoreType.DMA,
        pltpu.SemaphoreType.DMA,
        pltpu.SemaphoreType.DMA,
        pltpu.SemaphoreType.DMA,
        pltpu.SemaphoreType.DMA,
    ]
    k = pl.kernel(_sc_body, out_type=out_types, mesh=mesh,
                  scratch_types=scratch,
                  compiler_params=pltpu.CompilerParams(
                      needs_layout_passes=False))
    return k(dtq, dtc, tlq, tlc, llq, llc, wordv, docv, labv)


XB = 8192
XR = XB // 4


def _xpose_body(in_ref, out_ref):
    x = in_ref[...]
    stack = jnp.concatenate([x[:, p * XR:(p + 1) * XR] for p in range(4)],
                            axis=0)
    out_ref[...] = jnp.swapaxes(stack, 0, 1)


def _to_rowmajor(table):
    t_t = table.T
    v = t_t.shape[1]
    steps = -(-v // XB)
    return pl.pallas_call(
        _xpose_body,
        grid=(steps,),
        in_specs=[pl.BlockSpec((D, XB), lambda t: (0, t))],
        out_specs=pl.BlockSpec((XR, VW), lambda t: (t, 0)),
        out_shape=jax.ShapeDtypeStruct((steps * XR, VW), jnp.float32),
    )(t_t)


def _vq(i):
    return ((i >> 13) << 11) + (i & (XR - 1))


def _vc(i):
    return ((i >> 11) & 3) << 5


def _score_body(y_ref, x0_ref, lab_ref, out_ref):
    y = y_ref[...]
    lab = lab_ref[...]
    x0 = x0_ref[...]
    dn = (((1,), (1,)), ((), ()))
    s = lax.dot_general(y, lab, dn, preferred_element_type=jnp.float32)
    s0 = lax.dot_general(x0, lab, dn, preferred_element_type=jnp.float32)
    out_ref[...] = jnp.maximum(s - s0 + 1.0, 0.0)


def kernel(dt, tl, ll, num_sampled, opt, noise_ids, word_table, doc_table,
           label_table):
    del num_sampled, opt
    wordv = _to_rowmajor(word_table)
    docv = _to_rowmajor(doc_table)
    labv = _to_rowmajor(label_table)
    dtf = dt.reshape(-1)
    tlq2 = jnp.concatenate(
        [_vq(tl).reshape(NW, BW),
         jnp.broadcast_to(_vq(noise_ids), (NW, NSAMP))], axis=1).reshape(-1)
    tlc2 = jnp.concatenate(
        [_vc(tl).reshape(NW, BW),
         jnp.broadcast_to(_vc(noise_ids), (NW, NSAMP))], axis=1).reshape(-1)
    outdoc, labr, nzrows = _sc_call(
        _vq(dtf), _vc(dtf),
        tlq2, tlc2,
        _vq(ll), _vc(ll),
        wordv, docv, labv)
    outdoc = outdoc.reshape(B, D)
    labr = labr.reshape(B, D)
    nzrows = nzrows.reshape(NSAMP, D)
    y = jnp.concatenate([outdoc[1:], nzrows], axis=0)
    x0 = outdoc[0:1]
    return pl.pallas_call(
        _score_body,
        out_shape=jax.ShapeDtypeStruct((B + NSAMP - 1, B), jnp.float32),
    )(y, x0, labr)
